# baseline (speedup 1.0000x reference)
import jax
import jax.numpy as jnp
from jax.experimental import pallas as pl
from jax.experimental.pallas import tpu as pltpu

_N_BINS = 15
_ROWS, _COLS = 16384, 2048
_BLK = 512


def _fused_kernel(temp_ref, bc_ref, bt_ref, x_ref,
                  xout_ref, ece_ref, tout_ref, acc_ref):
    xout_ref[...] = x_ref[...]

    @pl.when(pl.program_id(0) == 0)
    def _stats():
        bc = bc_ref[...]
        bt = bt_ref[...]
        acc = bc / (bt + 1e-8)
        acc_ref[...] = acc
        lane_i = jax.lax.broadcasted_iota(jnp.int32, (1, 128), 1)
        mask = lane_i < _N_BINS
        lane = lane_i.astype(jnp.float32)
        conf = lane * (1.0 / (_N_BINS - 1)) + (0.5 / _N_BINS)
        n = jnp.sum(jnp.where(mask, bt, 0.0))
        contrib = jnp.where(mask, bt * jnp.abs(acc - conf), 0.0)
        ece = jnp.where(n > 0.0, jnp.sum(contrib) / jnp.maximum(n, 1e-8), 0.0)
        ece_ref[0, 0] = ece
        tout_ref[0, 0] = jnp.clip(temp_ref[0, 0], 0.1, 10.0)


def kernel(x, temperature, platt_a, platt_b, bin_correct, bin_total):
    bc = jnp.zeros((1, 128), jnp.float32).at[0, :_N_BINS].set(bin_correct)
    bt = jnp.zeros((1, 128), jnp.float32).at[0, :_N_BINS].set(bin_total)
    t2 = temperature.reshape(1, 1)
    xout, ece, temp, acc = pl.pallas_call(
        _fused_kernel,
        grid=(_ROWS // _BLK,),
        out_shape=(
            jax.ShapeDtypeStruct((_ROWS, _COLS), jnp.float32),
            jax.ShapeDtypeStruct((1, 1), jnp.float32),
            jax.ShapeDtypeStruct((1, 1), jnp.float32),
            jax.ShapeDtypeStruct((1, 128), jnp.float32),
        ),
        in_specs=[
            pl.BlockSpec(memory_space=pltpu.SMEM),
            pl.BlockSpec((1, 128), lambda i: (0, 0)),
            pl.BlockSpec((1, 128), lambda i: (0, 0)),
            pl.BlockSpec((_BLK, _COLS), lambda i: (i, 0)),
        ],
        out_specs=(
            pl.BlockSpec((_BLK, _COLS), lambda i: (i, 0)),
            pl.BlockSpec(memory_space=pltpu.SMEM),
            pl.BlockSpec(memory_space=pltpu.SMEM),
            pl.BlockSpec((1, 128), lambda i: (0, 0)),
        ),
    )(t2, bc, bt, x)
    return (xout, ece.reshape(()), temp.reshape(()), acc[0, :_N_BINS])


# speedup vs baseline: 1.0547x; 1.0547x over previous
import jax
import jax.numpy as jnp
from jax.experimental import pallas as pl
from jax.experimental.pallas import tpu as pltpu

_N_BINS = 15


def _stats_kernel(temp_ref, bc_ref, bt_ref, ece_ref, tout_ref, acc_ref):
    n = jnp.float32(0.0)
    for i in range(_N_BINS):
        n = n + bt_ref[i]
    s = jnp.float32(0.0)
    for i in range(_N_BINS):
        bc = bc_ref[i]
        bt = bt_ref[i]
        acc = bc / (bt + 1e-8)
        acc_ref[i] = acc
        conf = i / (_N_BINS - 1.0) + 0.5 / _N_BINS
        s = s + bt * jnp.abs(acc - conf)
    ece_ref[0] = jnp.where(n > 0.0, s / jnp.maximum(n, 1e-8), 0.0)
    tout_ref[0] = jnp.clip(temp_ref[0], 0.1, 10.0)


def kernel(x, temperature, platt_a, platt_b, bin_correct, bin_total):
    ece, temp, acc = pl.pallas_call(
        _stats_kernel,
        out_shape=(
            jax.ShapeDtypeStruct((1,), jnp.float32),
            jax.ShapeDtypeStruct((1,), jnp.float32),
            jax.ShapeDtypeStruct((_N_BINS,), jnp.float32),
        ),
        in_specs=[
            pl.BlockSpec(memory_space=pltpu.SMEM),
            pl.BlockSpec(memory_space=pltpu.SMEM),
            pl.BlockSpec(memory_space=pltpu.SMEM),
        ],
        out_specs=(
            pl.BlockSpec(memory_space=pltpu.SMEM),
            pl.BlockSpec(memory_space=pltpu.SMEM),
            pl.BlockSpec(memory_space=pltpu.SMEM),
        ),
    )(temperature.reshape(1), bin_correct, bin_total)
    return (x, ece.reshape(()), temp.reshape(()), acc)
